# Initial kernel scaffold; baseline (speedup 1.0000x reference)
import jax
import jax.numpy as jnp
from jax.experimental import pallas as pl


def kernel(node_attributes, edge_attributes, W_node, b_node, W_edge, b_edge, W_att, b_att, edge_node_indices):
    raise NotImplementedError("write your pallas kernel here")


# speedup vs baseline: 6.0640x; 6.0640x over previous
import functools

import jax
import jax.numpy as jnp
from jax import lax
from jax.experimental import pallas as pl
from jax.experimental.pallas import tpu as pltpu
from jax.experimental.pallas import tpu_sc as plsc

NC = 2
NS = 16
NW = NC * NS
CH = 128


def _node_linear(x, w, b):
    n, d = x.shape
    br = 1000
    grid = (n // br,)

    def body(x_ref, w_ref, b_ref, o_ref):
        o_ref[...] = lax.dot_general(
            x_ref[...], w_ref[...], (((1,), (1,)), ((), ())),
            preferred_element_type=jnp.float32) + b_ref[...]

    return pl.pallas_call(
        body,
        grid=grid,
        in_specs=[
            pl.BlockSpec((br, d), lambda i: (i, 0)),
            pl.BlockSpec((d, d), lambda i: (0, 0)),
            pl.BlockSpec((1, d), lambda i: (0, 0)),
        ],
        out_specs=pl.BlockSpec((br, d), lambda i: (i, 0)),
        out_shape=jax.ShapeDtypeStruct((n, d), jnp.float32),
    )(x, w, b.reshape(1, d))


def _combine(p0, p1):
    n, d = p0.shape
    br = 1000
    grid = (n // br,)

    def body(a_ref, b_ref, o_ref):
        o_ref[...] = a_ref[...] + b_ref[...]

    return pl.pallas_call(
        body,
        grid=grid,
        in_specs=[
            pl.BlockSpec((br, d), lambda i: (i, 0)),
            pl.BlockSpec((br, d), lambda i: (i, 0)),
        ],
        out_specs=pl.BlockSpec((br, d), lambda i: (i, 0)),
        out_shape=jax.ShapeDtypeStruct((n, d), jnp.float32),
    )(p0, p1)


def _scatter_partials(h, n0r, n1r, zrows, n_pad, steps):
    _, d = h.shape
    rps = n_pad // NS
    mesh = plsc.VectorSubcoreMesh(core_axis_name="c", subcore_axis_name="s")

    @functools.partial(
        pl.kernel,
        out_type=jax.ShapeDtypeStruct((NC, n_pad, d), jnp.float32),
        mesh=mesh,
        scratch_types=[
            pltpu.VMEM((CH,), jnp.int32),
            pltpu.VMEM((CH,), jnp.int32),
            pltpu.VMEM((CH, d), jnp.float32),
            pltpu.VMEM_SHARED((n_pad, d), jnp.float32),
            pltpu.SemaphoreType.DMA,
        ],
    )
    def k(h_hbm, n0_hbm, n1_hbm, z_hbm, out_hbm, idx0, idx1, rows, acc, sem):
        c = lax.axis_index("c")
        s = lax.axis_index("s")
        wid = c * NS + s
        pltpu.sync_copy(z_hbm, acc.at[pl.ds(s * rps, rps)])
        plsc.subcore_barrier()

        def step(i, carry):
            rb = wid * steps + i
            pltpu.sync_copy(n1_hbm.at[rb], idx1)
            pltpu.sync_copy(n0_hbm.at[rb], idx0)
            pltpu.async_copy(h_hbm.at[idx1], rows, sem).wait()
            pltpu.sync_copy(rows, acc.at[idx0], add=True)
            return carry

        lax.fori_loop(0, steps, step, 0)
        plsc.subcore_barrier()
        pltpu.sync_copy(acc.at[pl.ds(s * rps, rps)],
                        out_hbm.at[c].at[pl.ds(s * rps, rps)])

    return k(h, n0r, n1r, zrows)


def kernel(node_attributes, edge_attributes, W_node, b_node, W_edge, b_edge,
           W_att, b_att, edge_node_indices):
    n, d = node_attributes.shape
    e = edge_node_indices.shape[1]

    steps = -(-e // (NW * CH))
    e_pad = NW * steps * CH
    n_pad = -(-(n + 1) // (NS * 8)) * (NS * 8)

    n0 = edge_node_indices[0]
    n1 = edge_node_indices[1]
    pad = e_pad - e
    n0r = jnp.concatenate([n0, jnp.full((pad,), n, jnp.int32)]).reshape(-1, CH)
    n1r = jnp.concatenate([n1, jnp.zeros((pad,), jnp.int32)]).reshape(-1, CH)
    zrows = jnp.zeros((n_pad // NS, d), jnp.float32)

    h = _node_linear(node_attributes, W_node, b_node)
    partials = _scatter_partials(h, n0r, n1r, zrows, n_pad, steps)
    return _combine(partials[0, :n], partials[1, :n])
